# baseline (speedup 1.0000x reference)
import functools

import jax
import jax.numpy as jnp
from jax.experimental import pallas as pl

_K = 20
_NEG = -3.0e38
_EPS = 1e-5


def _gsa_conv_kernel(xf_ref, xb_ref, w_ref, y_ref, st_ref, *, kk):
    b = pl.program_id(0)
    i = pl.program_id(1)
    xf = xf_ref[0]
    xi = xb_ref[0]
    sq = jnp.sum(xf * xf, axis=0, keepdims=True)
    inner = jax.lax.dot_general(
        xi.astype(jnp.bfloat16), xf.astype(jnp.bfloat16), (((0,), (0,)), ((), ())),
        preferred_element_type=jnp.float32)
    a = 2.0 * inner - sq
    n = a.shape[1]
    l1 = a[:, 0:128]
    l2 = jnp.full_like(l1, _NEG)
    for c in range(1, n // 128):
        v = a[:, c * 128:(c + 1) * 128]
        l2 = jnp.maximum(l2, jnp.minimum(l1, v))
        l1 = jnp.maximum(l1, v)
    ll = jnp.concatenate([l1, l2], axis=1)
    m0 = jnp.max(l1, axis=1, keepdims=True)

    def body_ll(_, t):
        return jnp.max(jnp.where(ll < t, ll, _NEG), axis=1, keepdims=True)

    tc = jax.lax.fori_loop(0, kk - 1, body_ll, m0)
    cnt = jnp.sum(jnp.where(a >= tc, 1.0, 0.0), axis=1, keepdims=True)

    def full_scan(_):
        def body(_, t):
            return jnp.max(jnp.where(a < t, a, _NEG), axis=1, keepdims=True)
        return jax.lax.fori_loop(0, kk - 1, body, m0)

    t = jax.lax.cond(jnp.any(cnt != float(kk)), full_scan, lambda _: tc, 0)
    w = jnp.where(a >= t, jnp.exp(a - m0), 0.0)
    s = jnp.sum(w, axis=1, keepdims=True)
    agg = jax.lax.dot_general(
        xf, w, (((1,), (1,)), ((), ())),
        preferred_element_type=jnp.float32,
        precision=jax.lax.Precision.HIGHEST)
    rs = jnp.transpose(1.0 / s)
    fT = agg * rs - xi
    h = jnp.concatenate([xi, fT], axis=0)
    y = jax.lax.dot_general(
        w_ref[...], h.astype(jnp.bfloat16), (((1,), (0,)), ((), ())),
        preferred_element_type=jnp.float32)
    y_ref[0] = y

    @pl.when(jnp.logical_and(b == 0, i == 0))
    def _init():
        st_ref[...] = jnp.zeros_like(st_ref)

    ssum = jnp.sum(y, axis=1, keepdims=True)
    ssq = jnp.sum(y * y, axis=1, keepdims=True)
    st_ref[...] += jnp.concatenate([ssum, ssq], axis=1)


def _conv_kernel(h_ref, w_ref, y_ref, st_ref):
    b = pl.program_id(0)
    i = pl.program_id(1)
    y = jax.lax.dot_general(
        w_ref[...], h_ref[0].astype(jnp.bfloat16), (((1,), (0,)), ((), ())),
        preferred_element_type=jnp.float32)
    y_ref[0] = y

    @pl.when(jnp.logical_and(b == 0, i == 0))
    def _init():
        st_ref[...] = jnp.zeros_like(st_ref)

    ssum = jnp.sum(y, axis=1, keepdims=True)
    ssq = jnp.sum(y * y, axis=1, keepdims=True)
    st_ref[...] += jnp.concatenate([ssum, ssq], axis=1)


def _bn_act_kernel(y_ref, st_ref, gb_ref, o_ref, *, count):
    st = st_ref[...]
    mean = st[:, 0:1] / count
    var = st[:, 1:2] / count - mean * mean
    scale = gb_ref[:, 0:1] / jnp.sqrt(var + _EPS)
    bias = gb_ref[:, 1:2] - mean * scale
    z = y_ref[0] * scale + bias
    o_ref[0] = jnp.where(z >= 0.0, z, 0.01 * z)


def _gsa_conv(xin, W, bm):
    B, Dp, N = xin.shape
    O, twoD = W.shape
    D = twoD // 2
    WA, WB = W[:, :D], W[:, D:]
    if Dp != D:
        WA = jnp.pad(WA, ((0, 0), (0, Dp - D)))
        WB = jnp.pad(WB, ((0, 0), (0, Dp - D)))
    Wp = jnp.concatenate([WA, WB], axis=1).astype(jnp.bfloat16)
    nb = N // bm
    return pl.pallas_call(
        functools.partial(_gsa_conv_kernel, kk=_K),
        grid=(B, nb),
        in_specs=[
            pl.BlockSpec((1, Dp, N), lambda b, i: (b, 0, 0)),
            pl.BlockSpec((1, Dp, bm), lambda b, i: (b, 0, i)),
            pl.BlockSpec((O, 2 * Dp), lambda b, i: (0, 0)),
        ],
        out_specs=[
            pl.BlockSpec((1, O, bm), lambda b, i: (b, 0, i)),
            pl.BlockSpec((O, 2), lambda b, i: (0, 0)),
        ],
        out_shape=[
            jax.ShapeDtypeStruct((B, O, N), jnp.float32),
            jax.ShapeDtypeStruct((O, 2), jnp.float32),
        ],
    )(xin, xin, Wp)


def _conv5(h, W, bm):
    B, C, N = h.shape
    O = W.shape[0]
    return pl.pallas_call(
        _conv_kernel,
        grid=(B, N // bm),
        in_specs=[
            pl.BlockSpec((1, C, bm), lambda b, i: (b, 0, i)),
            pl.BlockSpec((O, C), lambda b, i: (0, 0)),
        ],
        out_specs=[
            pl.BlockSpec((1, O, bm), lambda b, i: (b, 0, i)),
            pl.BlockSpec((O, 2), lambda b, i: (0, 0)),
        ],
        out_shape=[
            jax.ShapeDtypeStruct((B, O, N), jnp.float32),
            jax.ShapeDtypeStruct((O, 2), jnp.float32),
        ],
    )(h, W.astype(jnp.bfloat16))


def _bn_act(y, st, g, bvec):
    B, O, N = y.shape
    gb = jnp.stack([g, bvec], axis=1)
    return pl.pallas_call(
        functools.partial(_bn_act_kernel, count=float(B * N)),
        grid=(B,),
        in_specs=[
            pl.BlockSpec((1, O, N), lambda b: (b, 0, 0)),
            pl.BlockSpec((O, 2), lambda b: (0, 0)),
            pl.BlockSpec((O, 2), lambda b: (0, 0)),
        ],
        out_specs=pl.BlockSpec((1, O, N), lambda b: (b, 0, 0)),
        out_shape=jax.ShapeDtypeStruct((B, O, N), jnp.float32),
    )(y, st, gb)


def kernel(x, W1, W2, W3, W4, W5, g1, b1, g2, b2, g3, b3, g4, b4, g5, b5):
    B, D0, N = x.shape
    bm = min(512, N)
    xp = jnp.pad(x, ((0, 0), (0, 8 - D0), (0, 0)))
    y1, st1 = _gsa_conv(xp, W1, bm)
    x1 = _bn_act(y1, st1, g1, b1)
    y2, st2 = _gsa_conv(x1, W2, bm)
    x2 = _bn_act(y2, st2, g2, b2)
    y3, st3 = _gsa_conv(x2, W3, bm)
    x3 = _bn_act(y3, st3, g3, b3)
    y4, st4 = _gsa_conv(x3, W4, bm)
    x4 = _bn_act(y4, st4, g4, b4)
    hcat = jnp.concatenate([x1, x2, x3, x4], axis=1)
    y5, st5 = _conv5(hcat, W5, bm)
    out = _bn_act(y5, st5, g5, b5)
    return (out, x3)


# speedup vs baseline: 1.6811x; 1.6811x over previous
import functools

import jax
import jax.numpy as jnp
from jax.experimental import pallas as pl

_K = 20
_NEG = -3.0e38
_EPS = 1e-5


def _gsa_conv_kernel(xf_ref, xb_ref, w_ref, y_ref, st_ref, *, kk):
    b = pl.program_id(0)
    i = pl.program_id(1)
    xf = xf_ref[0]
    xi = xb_ref[0]
    sq = jnp.sum(xf * xf, axis=0, keepdims=True)
    inner = jax.lax.dot_general(
        xi.astype(jnp.bfloat16), xf.astype(jnp.bfloat16), (((0,), (0,)), ((), ())),
        preferred_element_type=jnp.float32)
    a = 2.0 * inner - sq
    n = a.shape[1]
    l1 = a[:, 0:128]
    l2 = jnp.full_like(l1, _NEG)
    l3 = l2
    l4 = l2
    for c in range(1, n // 128):
        v = a[:, c * 128:(c + 1) * 128]
        lo = jnp.minimum(l1, v)
        l1 = jnp.maximum(l1, v)
        lo2 = jnp.minimum(l2, lo)
        l2 = jnp.maximum(l2, lo)
        lo3 = jnp.minimum(l3, lo2)
        l3 = jnp.maximum(l3, lo2)
        l4 = jnp.maximum(l4, lo3)
    ll = jnp.concatenate([l1, l2, l3, l4], axis=1)
    m0 = jnp.max(l1, axis=1, keepdims=True)

    def body_ll(_, t):
        return jnp.max(jnp.where(ll < t, ll, _NEG), axis=1, keepdims=True)

    tc = jax.lax.fori_loop(0, kk - 1, body_ll, m0)
    cnt = jnp.sum(jnp.where(a >= tc, 1.0, 0.0), axis=1, keepdims=True)

    def full_scan(_):
        def body(_, t):
            return jnp.max(jnp.where(a < t, a, _NEG), axis=1, keepdims=True)
        return jax.lax.fori_loop(0, kk - 1, body, m0)

    t = jax.lax.cond(jnp.any(cnt != float(kk)), full_scan, lambda _: tc, 0)
    w = jnp.where(a >= t, jnp.exp(a - m0), 0.0)
    s = jnp.sum(w, axis=1, keepdims=True)
    agg = jax.lax.dot_general(
        xf, w, (((1,), (1,)), ((), ())),
        preferred_element_type=jnp.float32,
        precision=jax.lax.Precision.HIGHEST)
    rs = jnp.transpose(1.0 / s)
    fT = agg * rs - xi
    h = jnp.concatenate([xi, fT], axis=0)
    y = jax.lax.dot_general(
        w_ref[...], h.astype(jnp.bfloat16), (((1,), (0,)), ((), ())),
        preferred_element_type=jnp.float32)
    y_ref[0] = y

    @pl.when(jnp.logical_and(b == 0, i == 0))
    def _init():
        st_ref[...] = jnp.zeros_like(st_ref)

    ssum = jnp.sum(y, axis=1, keepdims=True)
    ssq = jnp.sum(y * y, axis=1, keepdims=True)
    st_ref[...] += jnp.concatenate([ssum, ssq], axis=1)


def _conv_kernel(h_ref, w_ref, y_ref, st_ref):
    b = pl.program_id(0)
    i = pl.program_id(1)
    y = jax.lax.dot_general(
        w_ref[...], h_ref[0].astype(jnp.bfloat16), (((1,), (0,)), ((), ())),
        preferred_element_type=jnp.float32)
    y_ref[0] = y

    @pl.when(jnp.logical_and(b == 0, i == 0))
    def _init():
        st_ref[...] = jnp.zeros_like(st_ref)

    ssum = jnp.sum(y, axis=1, keepdims=True)
    ssq = jnp.sum(y * y, axis=1, keepdims=True)
    st_ref[...] += jnp.concatenate([ssum, ssq], axis=1)


def _bn_act_kernel(y_ref, st_ref, gb_ref, o_ref, *, count):
    st = st_ref[...]
    mean = st[:, 0:1] / count
    var = st[:, 1:2] / count - mean * mean
    scale = gb_ref[:, 0:1] / jnp.sqrt(var + _EPS)
    bias = gb_ref[:, 1:2] - mean * scale
    z = y_ref[0] * scale + bias
    o_ref[0] = jnp.where(z >= 0.0, z, 0.01 * z)


def _gsa_conv(xin, W, bm):
    B, Dp, N = xin.shape
    O, twoD = W.shape
    D = twoD // 2
    WA, WB = W[:, :D], W[:, D:]
    if Dp != D:
        WA = jnp.pad(WA, ((0, 0), (0, Dp - D)))
        WB = jnp.pad(WB, ((0, 0), (0, Dp - D)))
    Wp = jnp.concatenate([WA, WB], axis=1).astype(jnp.bfloat16)
    nb = N // bm
    return pl.pallas_call(
        functools.partial(_gsa_conv_kernel, kk=_K),
        grid=(B, nb),
        in_specs=[
            pl.BlockSpec((1, Dp, N), lambda b, i: (b, 0, 0)),
            pl.BlockSpec((1, Dp, bm), lambda b, i: (b, 0, i)),
            pl.BlockSpec((O, 2 * Dp), lambda b, i: (0, 0)),
        ],
        out_specs=[
            pl.BlockSpec((1, O, bm), lambda b, i: (b, 0, i)),
            pl.BlockSpec((O, 2), lambda b, i: (0, 0)),
        ],
        out_shape=[
            jax.ShapeDtypeStruct((B, O, N), jnp.float32),
            jax.ShapeDtypeStruct((O, 2), jnp.float32),
        ],
    )(xin, xin, Wp)


def _conv5(h, W, bm):
    B, C, N = h.shape
    O = W.shape[0]
    return pl.pallas_call(
        _conv_kernel,
        grid=(B, N // bm),
        in_specs=[
            pl.BlockSpec((1, C, bm), lambda b, i: (b, 0, i)),
            pl.BlockSpec((O, C), lambda b, i: (0, 0)),
        ],
        out_specs=[
            pl.BlockSpec((1, O, bm), lambda b, i: (b, 0, i)),
            pl.BlockSpec((O, 2), lambda b, i: (0, 0)),
        ],
        out_shape=[
            jax.ShapeDtypeStruct((B, O, N), jnp.float32),
            jax.ShapeDtypeStruct((O, 2), jnp.float32),
        ],
    )(h, W.astype(jnp.bfloat16))


def _bn_act(y, st, g, bvec):
    B, O, N = y.shape
    gb = jnp.stack([g, bvec], axis=1)
    return pl.pallas_call(
        functools.partial(_bn_act_kernel, count=float(B * N)),
        grid=(B,),
        in_specs=[
            pl.BlockSpec((1, O, N), lambda b: (b, 0, 0)),
            pl.BlockSpec((O, 2), lambda b: (0, 0)),
            pl.BlockSpec((O, 2), lambda b: (0, 0)),
        ],
        out_specs=pl.BlockSpec((1, O, N), lambda b: (b, 0, 0)),
        out_shape=jax.ShapeDtypeStruct((B, O, N), jnp.float32),
    )(y, st, gb)


def kernel(x, W1, W2, W3, W4, W5, g1, b1, g2, b2, g3, b3, g4, b4, g5, b5):
    B, D0, N = x.shape
    bm = min(512, N)
    xp = jnp.pad(x, ((0, 0), (0, 8 - D0), (0, 0)))
    y1, st1 = _gsa_conv(xp, W1, bm)
    x1 = _bn_act(y1, st1, g1, b1)
    y2, st2 = _gsa_conv(x1, W2, bm)
    x2 = _bn_act(y2, st2, g2, b2)
    y3, st3 = _gsa_conv(x2, W3, bm)
    x3 = _bn_act(y3, st3, g3, b3)
    y4, st4 = _gsa_conv(x3, W4, bm)
    x4 = _bn_act(y4, st4, g4, b4)
    hcat = jnp.concatenate([x1, x2, x3, x4], axis=1)
    y5, st5 = _conv5(hcat, W5, bm)
    out = _bn_act(y5, st5, g5, b5)
    return (out, x3)


# speedup vs baseline: 1.8264x; 1.0864x over previous
import functools

import jax
import jax.numpy as jnp
from jax.experimental import pallas as pl

_K = 20
_NEG = -3.0e38
_EPS = 1e-5


def _gsa_conv_kernel(xf_ref, xb_ref, w_ref, y_ref, st_ref, *, kk):
    b = pl.program_id(0)
    i = pl.program_id(1)
    xfa = xf_ref[0]
    xia = xb_ref[0]
    dp = xfa.shape[0] - 8
    xf = xfa[0:dp]
    xi = xia[0:dp]
    sq = jnp.sum(xf * xf, axis=0, keepdims=True)
    inner = jax.lax.dot_general(
        xi.astype(jnp.bfloat16), xf.astype(jnp.bfloat16), (((0,), (0,)), ((), ())),
        preferred_element_type=jnp.float32)
    a = 2.0 * inner - sq
    n = a.shape[1]
    l1 = a[:, 0:128]
    l2 = jnp.full_like(l1, _NEG)
    l3 = l2
    l4 = l2
    l5 = l2
    for c in range(1, n // 128):
        v = a[:, c * 128:(c + 1) * 128]
        lo = jnp.minimum(l1, v)
        l1 = jnp.maximum(l1, v)
        lo2 = jnp.minimum(l2, lo)
        l2 = jnp.maximum(l2, lo)
        lo3 = jnp.minimum(l3, lo2)
        l3 = jnp.maximum(l3, lo2)
        lo4 = jnp.minimum(l4, lo3)
        l4 = jnp.maximum(l4, lo3)
        l5 = jnp.maximum(l5, lo4)
    ll = jnp.concatenate([l1, l2, l3, l4], axis=1)
    m0 = jnp.max(l1, axis=1, keepdims=True)

    def body_ll(_, t):
        return jnp.max(jnp.where(ll < t, ll, _NEG), axis=1, keepdims=True)

    tc = jax.lax.fori_loop(0, kk - 1, body_ll, m0)

    def full_scan(_):
        def body(_, t):
            return jnp.max(jnp.where(a < t, a, _NEG), axis=1, keepdims=True)
        return jax.lax.fori_loop(0, kk - 1, body, m0)

    t = jax.lax.cond(jnp.any(l5 >= tc), full_scan, lambda _: tc, 0)
    w = jnp.where(a >= t, jnp.exp(a - m0), 0.0)
    agg = jax.lax.dot_general(
        xfa, w, (((1,), (1,)), ((), ())),
        preferred_element_type=jnp.float32,
        precision=jax.lax.Precision.HIGHEST)
    rs = 1.0 / agg[dp:dp + 1, :]
    fT = agg[0:dp] * rs - xi
    h = jnp.concatenate([xi, fT], axis=0)
    y = jax.lax.dot_general(
        w_ref[...], h.astype(jnp.bfloat16), (((1,), (0,)), ((), ())),
        preferred_element_type=jnp.float32)
    y_ref[0] = y

    @pl.when(jnp.logical_and(b == 0, i == 0))
    def _init():
        st_ref[...] = jnp.zeros_like(st_ref)

    ssum = jnp.sum(y, axis=1, keepdims=True)
    ssq = jnp.sum(y * y, axis=1, keepdims=True)
    st_ref[...] += jnp.concatenate([ssum, ssq], axis=1)


def _conv_kernel(h_ref, w_ref, y_ref, st_ref):
    b = pl.program_id(0)
    i = pl.program_id(1)
    y = jax.lax.dot_general(
        w_ref[...], h_ref[0].astype(jnp.bfloat16), (((1,), (0,)), ((), ())),
        preferred_element_type=jnp.float32)
    y_ref[0] = y

    @pl.when(jnp.logical_and(b == 0, i == 0))
    def _init():
        st_ref[...] = jnp.zeros_like(st_ref)

    ssum = jnp.sum(y, axis=1, keepdims=True)
    ssq = jnp.sum(y * y, axis=1, keepdims=True)
    st_ref[...] += jnp.concatenate([ssum, ssq], axis=1)


def _bn_act_kernel(y_ref, st_ref, gb_ref, o_ref, *, count):
    st = st_ref[...]
    mean = st[:, 0:1] / count
    var = st[:, 1:2] / count - mean * mean
    scale = gb_ref[:, 0:1] / jnp.sqrt(var + _EPS)
    bias = gb_ref[:, 1:2] - mean * scale
    z = y_ref[0] * scale + bias
    o_ref[0] = jnp.where(z >= 0.0, z, 0.01 * z)


def _gsa_conv(xin, W, bm):
    B, Dp, N = xin.shape
    O, twoD = W.shape
    D = twoD // 2
    WA, WB = W[:, :D], W[:, D:]
    if Dp != D:
        WA = jnp.pad(WA, ((0, 0), (0, Dp - D)))
        WB = jnp.pad(WB, ((0, 0), (0, Dp - D)))
    Wp = jnp.concatenate([WA, WB], axis=1).astype(jnp.bfloat16)
    xaug = jnp.concatenate(
        [xin, jnp.ones((B, 1, N), jnp.float32),
         jnp.zeros((B, 7, N), jnp.float32)], axis=1)
    Da = Dp + 8
    nb = N // bm
    return pl.pallas_call(
        functools.partial(_gsa_conv_kernel, kk=_K),
        grid=(B, nb),
        in_specs=[
            pl.BlockSpec((1, Da, N), lambda b, i: (b, 0, 0)),
            pl.BlockSpec((1, Da, bm), lambda b, i: (b, 0, i)),
            pl.BlockSpec((O, 2 * Dp), lambda b, i: (0, 0)),
        ],
        out_specs=[
            pl.BlockSpec((1, O, bm), lambda b, i: (b, 0, i)),
            pl.BlockSpec((O, 2), lambda b, i: (0, 0)),
        ],
        out_shape=[
            jax.ShapeDtypeStruct((B, O, N), jnp.float32),
            jax.ShapeDtypeStruct((O, 2), jnp.float32),
        ],
    )(xaug, xaug, Wp)


def _conv5(h, W, bm):
    B, C, N = h.shape
    O = W.shape[0]
    return pl.pallas_call(
        _conv_kernel,
        grid=(B, N // bm),
        in_specs=[
            pl.BlockSpec((1, C, bm), lambda b, i: (b, 0, i)),
            pl.BlockSpec((O, C), lambda b, i: (0, 0)),
        ],
        out_specs=[
            pl.BlockSpec((1, O, bm), lambda b, i: (b, 0, i)),
            pl.BlockSpec((O, 2), lambda b, i: (0, 0)),
        ],
        out_shape=[
            jax.ShapeDtypeStruct((B, O, N), jnp.float32),
            jax.ShapeDtypeStruct((O, 2), jnp.float32),
        ],
    )(h, W.astype(jnp.bfloat16))


def _bn_act(y, st, g, bvec):
    B, O, N = y.shape
    gb = jnp.stack([g, bvec], axis=1)
    return pl.pallas_call(
        functools.partial(_bn_act_kernel, count=float(B * N)),
        grid=(B,),
        in_specs=[
            pl.BlockSpec((1, O, N), lambda b: (b, 0, 0)),
            pl.BlockSpec((O, 2), lambda b: (0, 0)),
            pl.BlockSpec((O, 2), lambda b: (0, 0)),
        ],
        out_specs=pl.BlockSpec((1, O, N), lambda b: (b, 0, 0)),
        out_shape=jax.ShapeDtypeStruct((B, O, N), jnp.float32),
    )(y, st, gb)


def kernel(x, W1, W2, W3, W4, W5, g1, b1, g2, b2, g3, b3, g4, b4, g5, b5):
    B, D0, N = x.shape
    bm = min(512, N)
    xp = jnp.pad(x, ((0, 0), (0, 8 - D0), (0, 0)))
    y1, st1 = _gsa_conv(xp, W1, bm)
    x1 = _bn_act(y1, st1, g1, b1)
    y2, st2 = _gsa_conv(x1, W2, bm)
    x2 = _bn_act(y2, st2, g2, b2)
    y3, st3 = _gsa_conv(x2, W3, bm)
    x3 = _bn_act(y3, st3, g3, b3)
    y4, st4 = _gsa_conv(x3, W4, bm)
    x4 = _bn_act(y4, st4, g4, b4)
    hcat = jnp.concatenate([x1, x2, x3, x4], axis=1)
    y5, st5 = _conv5(hcat, W5, bm)
    out = _bn_act(y5, st5, g5, b5)
    return (out, x3)
